# Initial kernel scaffold; baseline (speedup 1.0000x reference)
import jax
import jax.numpy as jnp
from jax.experimental import pallas as pl


def kernel(vertices, faces, K, total_num, delta):
    raise NotImplementedError("write your pallas kernel here")


# speedup vs baseline: 1.0232x; 1.0232x over previous
import jax
import jax.numpy as jnp
from jax.experimental import pallas as pl
from jax.experimental.pallas import tpu as pltpu

V = 100000
F = 200000

_CBV = 30720
_CBF = 61440
_GRID = 10


def _copy_body(offf_ref, offi_ref, vf_ref, ff_ref, vrep_ref, frep_ref):
    offf = offf_ref[0, 0]
    offi = offi_ref[0, 0]
    vrow = vf_ref[...] + offf
    frow = ff_ref[...] + offi
    vrep_ref[...] = jnp.broadcast_to(vrow, (4,) + vrow.shape)
    frep_ref[...] = jnp.broadcast_to(frow, (4,) + frow.shape)


def _tile_outputs(v_flat, f_flat, off_f, off_i):
    vrep_flat, frep_flat = pl.pallas_call(
        _copy_body,
        grid=(_GRID,),
        in_specs=[
            pl.BlockSpec((1, 1), lambda i: (0, 0)),
            pl.BlockSpec((1, 1), lambda i: (0, 0)),
            pl.BlockSpec((_CBV,), lambda i: (i,)),
            pl.BlockSpec((_CBF,), lambda i: (i,)),
        ],
        out_specs=[
            pl.BlockSpec((4, _CBV), lambda i: (0, i)),
            pl.BlockSpec((4, _CBF), lambda i: (0, i)),
        ],
        out_shape=[
            jax.ShapeDtypeStruct((4, 3 * V), jnp.float32),
            jax.ShapeDtypeStruct((4, 3 * F), jnp.int32),
        ],
    )(off_f, off_i, v_flat, f_flat)
    return vrep_flat, frep_flat


def kernel(vertices, faces, K, total_num, delta):
    off = jnp.asarray(total_num) - 4
    off_f = off.astype(jnp.float32).reshape(1, 1)
    off_i = off.astype(jnp.int32).reshape(1, 1)

    src = jnp.concatenate([faces[:, 0], faces[:, 1], faces[:, 2],
                           faces[:, 1], faces[:, 2], faces[:, 0]])
    dst = jnp.concatenate([faces[:, 1], faces[:, 2], faces[:, 0],
                           faces[:, 0], faces[:, 1], faces[:, 2]])
    ones = jnp.ones(src.shape[0], dtype=jnp.float32)
    deg = jax.ops.segment_sum(ones, src, num_segments=V)
    nbr_sum = jax.ops.segment_sum(jnp.take(vertices, dst, axis=0), src, num_segments=V)
    lap = nbr_sum / jnp.maximum(deg, 1.0)[:, None] - vertices
    laplacian_loss = jnp.mean(jnp.linalg.norm(lap, axis=1))
    hexagon_loss = jnp.mean(K * jnp.sum(lap * lap, axis=1))

    vrep_flat, frep_flat = _tile_outputs(
        vertices.reshape(-1), faces.reshape(-1), off_f, off_i)
    v_rep = vrep_flat.reshape(4, V, 3)
    f_rep = frep_flat.reshape(4, F, 3)
    zero = jnp.float32(0.0)
    return (v_rep, f_rep, laplacian_loss, hexagon_loss, zero, zero)


# speedup vs baseline: 6.3582x; 6.2139x over previous
import functools

import jax
import jax.numpy as jnp
from jax import lax
from jax.experimental import pallas as pl
from jax.experimental.pallas import tpu as pltpu
from jax.experimental.pallas import tpu_sc as plsc

V = 100000
F = 200000

NC = 2
NS = 16
LANES = 16

VP = 100352
FP = 200704
FW = FP // (NC * NS)
FH = FW // 2
NCHUNK = FW // 128
VROWS_T = VP // NS
VROWS_W = VP // (NC * NS)

_mesh = plsc.VectorSubcoreMesh(
    core_axis_name="c", subcore_axis_name="s", num_cores=NC, num_subcores=NS)


def _scatter_body(vpad, a3, b3, c3, zrows, out,
                  ja, jb, jc, ra, rb, rc, acc, sa, sb, sc):
    c = lax.axis_index("c")
    s = lax.axis_index("s")
    w = c * NS + s

    pltpu.sync_copy(zrows, acc.at[pl.ds(s * VROWS_T, VROWS_T)])

    pltpu.sync_copy(a3.at[w], ja)
    pltpu.sync_copy(b3.at[w], jb)
    pltpu.sync_copy(c3.at[w], jc)

    plsc.subcore_barrier()

    def step(k, _):
        da = pltpu.async_copy(vpad.at[ja.at[k]], ra, sa)
        db = pltpu.async_copy(vpad.at[jb.at[k]], rb, sb)
        dc = pltpu.async_copy(vpad.at[jc.at[k]], rc, sc)
        da.wait()
        db.wait()
        dc.wait()
        for vals, i2 in ((rb, ja), (rc, ja), (ra, jb),
                         (rc, jb), (ra, jc), (rb, jc)):
            pltpu.sync_copy(vals, acc.at[i2.at[k]], add=True)
        return 0

    lax.fori_loop(0, NCHUNK, step, 0)

    plsc.subcore_barrier()
    pltpu.sync_copy(acc.at[pl.ds(s * VROWS_T, VROWS_T)],
                    out.at[c, pl.ds(s * VROWS_T, VROWS_T)])


_scatter_call = functools.partial(
    pl.kernel,
    out_type=jax.ShapeDtypeStruct((NC, VP, 8), jnp.float32),
    mesh=_mesh,
    scratch_types=[
        pltpu.VMEM((NCHUNK, 128), jnp.int32),
        pltpu.VMEM((NCHUNK, 128), jnp.int32),
        pltpu.VMEM((NCHUNK, 128), jnp.int32),
        pltpu.VMEM((128, 8), jnp.float32),
        pltpu.VMEM((128, 8), jnp.float32),
        pltpu.VMEM((128, 8), jnp.float32),
        pltpu.VMEM_SHARED((VP, 8), jnp.float32),
        pltpu.SemaphoreType.DMA,
        pltpu.SemaphoreType.DMA,
        pltpu.SemaphoreType.DMA,
    ],
    compiler_params=pltpu.CompilerParams(
        use_tc_tiling_on_sc=False, needs_layout_passes=False),
)


_DNUMS = lax.GatherDimensionNumbers(
    offset_dims=(), collapsed_slice_dims=(0,), start_index_map=(0,))


def _dg(x, idx):
    return lax.gather(x, idx[:, None], _DNUMS, (1,),
                      mode=lax.GatherScatterMode.PROMISE_IN_BOUNDS)


def _loss_body(parts, vflat, kp, out,
               pv0, pv1, vv, kv, pr, pall, orp, shp):
    c = lax.axis_index("c")
    s = lax.axis_index("s")
    w = c * NS + s
    base = w * VROWS_W

    pltpu.sync_copy(parts.at[pl.ds(base * 8, VROWS_W * 8)], pv0)
    pltpu.sync_copy(parts.at[pl.ds(VP * 8 + base * 8, VROWS_W * 8)], pv1)
    pltpu.sync_copy(vflat.at[pl.ds(base * 8, VROWS_W * 8)], vv)
    pltpu.sync_copy(kp.at[pl.ds(base, VROWS_W)], kv)

    iv = lax.iota(jnp.int32, LANES)
    q = lax.shift_right_logical(iv, 3)
    cc = jnp.bitwise_and(iv, 7)
    mgood = cc < 3
    m0 = cc == 0
    dgi = jnp.bitwise_or(jnp.bitwise_and(iv, 8), 3)
    x1 = jnp.bitwise_xor(iv, 1)
    x2 = jnp.bitwise_xor(iv, 2)
    x4 = jnp.bitwise_xor(iv, 4)
    zv = jnp.zeros((LANES,), jnp.float32)

    def group(t, carry):
        al, ah = carry
        k16 = kv[pl.ds(t * LANES, LANES)]
        for u in range(8):
            i = 8 * t + u
            a = pv0[pl.ds(i * LANES, LANES)] + pv1[pl.ds(i * LANES, LANES)]
            vvv = vv[pl.ds(i * LANES, LANES)]
            deg = _dg(a, dgi)
            lap = a / jnp.maximum(deg, 1.0) - vvv
            lap = jnp.where(mgood, lap, 0.0)
            sq = lap * lap
            s1 = sq + _dg(sq, x1)
            s2 = s1 + _dg(s1, x2)
            s3 = s2 + _dg(s2, x4)
            yi = plsc.bitcast(s3, jnp.int32)
            yi = 0x5F3759DF - lax.shift_right_logical(yi, 1)
            y = plsc.bitcast(yi, jnp.float32)
            for _ in range(3):
                y = y * (1.5 - 0.5 * s3 * y * y)
            nrm = jnp.where(s3 > 0, s3 * y, 0.0)
            kq = _dg(k16, 2 * u + q)
            al = al + jnp.where(m0, nrm, zv)
            ah = ah + jnp.where(m0, kq * s3, zv)
        return al, ah

    al, ah = lax.fori_loop(0, VROWS_W // LANES, group, (zv, zv))
    pr[pl.ds(0, LANES)] = al
    pr[pl.ds(LANES, LANES)] = ah
    pltpu.sync_copy(pr, shp.at[pl.ds(s * 32, 32)])
    plsc.subcore_barrier()

    @pl.when(s == 0)
    def _():
        pltpu.sync_copy(shp, pall)
        tl = zv
        th = zv
        for j in range(NS):
            tl = tl + pall[pl.ds(j * 32, LANES)]
            th = th + pall[pl.ds(j * 32 + LANES, LANES)]
        inv = jnp.float32(1.0 / V)
        orp[pl.ds(0, LANES)] = tl * inv
        orp[pl.ds(LANES, LANES)] = th * inv
        pltpu.sync_copy(orp, out.at[pl.ds(c * 32, 32)])


_loss_call = functools.partial(
    pl.kernel,
    out_type=jax.ShapeDtypeStruct((NC * 32,), jnp.float32),
    mesh=_mesh,
    scratch_types=[
        pltpu.VMEM((VROWS_W * 8,), jnp.float32),
        pltpu.VMEM((VROWS_W * 8,), jnp.float32),
        pltpu.VMEM((VROWS_W * 8,), jnp.float32),
        pltpu.VMEM((VROWS_W,), jnp.float32),
        pltpu.VMEM((32,), jnp.float32),
        pltpu.VMEM((NS * 32,), jnp.float32),
        pltpu.VMEM((32,), jnp.float32),
        pltpu.VMEM_SHARED((NS * 32,), jnp.float32),
    ],
    compiler_params=pltpu.CompilerParams(needs_layout_passes=False),
)

_CBV = 30720
_CBF = 61440
_GRID = 10


def _copy_body(offf_ref, offi_ref, vf_ref, ff_ref, vrep_ref, frep_ref):
    offf = offf_ref[0, 0]
    offi = offi_ref[0, 0]
    vrow = vf_ref[...] + offf
    frow = ff_ref[...] + offi
    vrep_ref[...] = jnp.broadcast_to(vrow, (4,) + vrow.shape)
    frep_ref[...] = jnp.broadcast_to(frow, (4,) + frow.shape)


def _tile_outputs(v_flat, f_flat, off_f, off_i):
    return pl.pallas_call(
        _copy_body,
        grid=(_GRID,),
        in_specs=[
            pl.BlockSpec((1, 1), lambda i: (0, 0)),
            pl.BlockSpec((1, 1), lambda i: (0, 0)),
            pl.BlockSpec((_CBV,), lambda i: (i,)),
            pl.BlockSpec((_CBF,), lambda i: (i,)),
        ],
        out_specs=[
            pl.BlockSpec((4, _CBV), lambda i: (0, i)),
            pl.BlockSpec((4, _CBF), lambda i: (0, i)),
        ],
        out_shape=[
            jax.ShapeDtypeStruct((4, 3 * V), jnp.float32),
            jax.ShapeDtypeStruct((4, 3 * F), jnp.int32),
        ],
    )(off_f, off_i, v_flat, f_flat)


def kernel(vertices, faces, K, total_num, delta):
    off = jnp.asarray(total_num) - 4
    off_f = off.astype(jnp.float32).reshape(1, 1)
    off_i = off.astype(jnp.int32).reshape(1, 1)

    vpad = jnp.pad(vertices, ((0, VP - V), (0, 5)))
    vpad = vpad.at[:, 3].set(1.0)
    kpad = jnp.pad(K, (0, VP - V))
    npadf = FP - F
    padv = (V + jnp.arange(npadf, dtype=jnp.int32) % (VP - V - 8))
    col3 = [jnp.concatenate([faces[:, j].astype(jnp.int32), padv])
            .reshape(NC * NS, NCHUNK, 128) for j in range(3)]
    zrows = jnp.zeros((VROWS_T, 8), jnp.float32)

    parts = _scatter_call(_scatter_body)(
        vpad, col3[0], col3[1], col3[2], zrows)
    sums = _loss_call(_loss_body)(
        parts.reshape(NC * VP * 8), vpad.reshape(VP * 8), kpad)
    sums = sums.reshape(NC, 32)
    laplacian_loss = jnp.sum(sums[:, :LANES])
    hexagon_loss = jnp.sum(sums[:, LANES:])

    vrep_flat, frep_flat = _tile_outputs(
        vertices.reshape(-1), faces.reshape(-1), off_f, off_i)
    v_rep = vrep_flat.reshape(4, V, 3)
    f_rep = frep_flat.reshape(4, F, 3)
    zero = jnp.float32(0.0)
    return (v_rep, f_rep, laplacian_loss, hexagon_loss, zero, zero)
